# baseline (speedup 1.0000x reference)
import functools

import jax
import jax.numpy as jnp
from jax import lax
from jax.experimental import pallas as pl
from jax.experimental.pallas import tpu as pltpu
from jax.experimental.pallas import tpu_sc as plsc

_NC = 2
_NS = 16
_NW = _NC * _NS
_L = 16

_N = 10000
_E = 320000
_D = 128

_CH = 128
_NSUB = 4
_KP = 32
_NPH0 = 3
_NPH1 = 2
_K0 = _NPH0 * _KP
_K1 = _NPH1 * _KP
_TOTC = _NS * (_K0 + _K1)
_EPT = (_K0 + _K1) * _CH
_EPAD = _TOTC * _CH
_EPD = _EPAD // _NW
_ACC = 10240
_RPT = _ACC // _NS

_mesh = plsc.VectorSubcoreMesh(
    core_axis_name="c", subcore_axis_name="s", num_cores=_NC, num_subcores=_NS
)


@functools.partial(
    pl.kernel,
    out_type=jax.ShapeDtypeStruct((_NW, _ACC), jnp.float32),
    mesh=_mesh,
    scratch_types=[
        pltpu.VMEM((_EPD,), jnp.int32),
        pltpu.VMEM((_ACC,), jnp.float32),
    ],
    compiler_params=pltpu.CompilerParams(needs_layout_passes=False),
)
def _deg_kernel(dst_hbm, out_hbm, dstv, degv):
    c = lax.axis_index("c")
    s = lax.axis_index("s")
    w = c * _NS + s
    pltpu.sync_copy(dst_hbm.at[pl.ds(w * _EPD, _EPD)], dstv)

    zeros = jnp.zeros((_L,), jnp.float32)

    @pl.loop(0, _ACC // _L)
    def _zero(i):
        degv[pl.ds(i * _L, _L)] = zeros

    ones = jnp.ones((_L,), jnp.float32)

    @pl.loop(0, _EPD // _L)
    def _hist(i):
        idx = dstv[pl.ds(i * _L, _L)]
        plsc.addupdate_scatter(degv, [idx], ones)

    pltpu.sync_copy(degv, out_hbm.at[w])


def _dis_body(parts_ref, dis_ref):
    deg = jnp.sum(parts_ref[...], axis=0) + 1.0
    dis_ref[...] = lax.rsqrt(deg)


def _dis_kernel(parts):
    return pl.pallas_call(
        _dis_body,
        out_shape=jax.ShapeDtypeStruct((_ACC // 128, 128), jnp.float32),
    )(parts.reshape(_NW, _ACC // 128, 128))


def _matmul_body(x_ref, w_ref, dis_ref, g_ref):
    h = lax.dot_general(
        x_ref[...], w_ref[...],
        dimension_numbers=(((1,), (1,)), ((), ())),
        preferred_element_type=jnp.float32,
    )
    g_ref[...] = h * dis_ref[...]


def _matmul_kernel(x, w, dis):
    bm = 1000
    return pl.pallas_call(
        _matmul_body,
        grid=(_N // bm,),
        in_specs=[
            pl.BlockSpec((bm, _D), lambda i: (i, 0)),
            pl.BlockSpec((_D, _D), lambda i: (0, 0)),
            pl.BlockSpec((bm, 1), lambda i: (i, 0)),
        ],
        out_specs=pl.BlockSpec((bm, _D), lambda i: (i, 0)),
        out_shape=jax.ShapeDtypeStruct((_N, _D), jnp.float32),
    )(x, w, dis)


@functools.partial(
    pl.kernel,
    out_type=jax.ShapeDtypeStruct((_NC, _ACC, _D), jnp.float32),
    mesh=_mesh,
    scratch_types=[
        pltpu.VMEM((_KP, _CH), jnp.int32),
        pltpu.VMEM((_KP, _CH), jnp.int32),

        pltpu.VMEM((_CH, _D), jnp.float32),
        pltpu.VMEM((_CH, _D), jnp.float32),
        pltpu.VMEM_SHARED((_ACC, _D), jnp.float32),
        pltpu.SemaphoreType.DMA,
        pltpu.SemaphoreType.DMA,
    ],
)
def _scatter_kernel(g_hbm, src_hbm, dst_hbm, out_hbm,
                    srcv, dstv, buf0, buf1, acc, sem0, sem1):
    c = lax.axis_index("c")
    s = lax.axis_index("s")
    base = jnp.where(c == 0, s * _K0, _NS * _K0 + s * _K1)
    nph = jnp.where(c == 0, _NPH0, _NPH1)

    zeros = jnp.zeros((_L,), jnp.float32)

    @pl.loop(0, _CH * _D // _L)
    def _zero(i):
        r = i >> 3
        col = (i & 7) * _L
        buf0[r, pl.ds(col, _L)] = zeros

    @pl.loop(0, _RPT // _CH)
    def _zacc(i):
        pltpu.sync_copy(buf0, acc.at[pl.ds(s * _RPT + i * _CH, _CH)])

    plsc.subcore_barrier()

    def _start(j, buf, sem):
        sub = _CH // _NSUB
        for q in range(_NSUB):
            pltpu.async_copy(
                g_hbm.at[srcv.at[j, pl.ds(q * sub, sub)]],
                buf.at[pl.ds(q * sub, sub)],
                sem,
            )

    def _wait(buf, sem):
        pltpu.make_async_copy(g_hbm.at[srcv.at[0]], buf, sem).wait()

    @pl.loop(0, nph)
    def _phase(p):
        pltpu.sync_copy(src_hbm.at[pl.ds(base + p * _KP, _KP)], srcv)
        pltpu.sync_copy(dst_hbm.at[pl.ds(base + p * _KP, _KP)], dstv)

        _start(0, buf0, sem0)
        _start(1, buf1, sem1)

        @pl.loop(0, _KP // 2)
        def _edges(i):
            j0 = 2 * i
            _wait(buf0, sem0)
            pltpu.sync_copy(buf0, acc.at[dstv.at[j0]], add=True)

            @pl.when(j0 + 2 < _KP)
            def _():
                _start(j0 + 2, buf0, sem0)

            j1 = 2 * i + 1
            _wait(buf1, sem1)
            pltpu.sync_copy(buf1, acc.at[dstv.at[j1]], add=True)

            @pl.when(j1 + 2 < _KP)
            def _():
                _start(j1 + 2, buf1, sem1)

    plsc.subcore_barrier()
    pltpu.sync_copy(acc.at[pl.ds(s * _RPT, _RPT)],
                    out_hbm.at[c, pl.ds(s * _RPT, _RPT)])


def _final_body(p0_ref, p1_ref, g_ref, dis_ref, b_ref, out_ref):
    out_ref[...] = (
        dis_ref[...] * (p0_ref[...] + p1_ref[...] + g_ref[...]) + b_ref[...]
    )


def _final_kernel(p0, p1, g, dis, b):
    bm = 1000
    row = pl.BlockSpec((bm, _D), lambda i: (i, 0))
    return pl.pallas_call(
        _final_body,
        grid=(_N // bm,),
        in_specs=[
            row, row, row,
            pl.BlockSpec((bm, 1), lambda i: (i, 0)),
            pl.BlockSpec((1, _D), lambda i: (0, 0)),
        ],
        out_specs=row,
        out_shape=jax.ShapeDtypeStruct((_N, _D), jnp.float32),
    )(p0, p1, g, dis, b)


def kernel(x, edge_index, W, b):
    src = edge_index[0].astype(jnp.int32)
    dst = edge_index[1].astype(jnp.int32)
    npad = _EPAD - _E
    src_p = jnp.concatenate([src, jnp.zeros((npad,), jnp.int32)])
    dst_p = jnp.concatenate([dst, jnp.full((npad,), _N, jnp.int32)])

    deg_parts = _deg_kernel(dst_p)
    dis2d = _dis_kernel(deg_parts)
    dis = dis2d.reshape(-1)[:_N, None]
    g = _matmul_kernel(x, W, dis)
    parts = _scatter_kernel(
        g, src_p.reshape(_TOTC, _CH), dst_p.reshape(_TOTC, _CH)
    )
    out = _final_kernel(parts[0, :_N], parts[1, :_N], g, dis, b.reshape(1, _D))
    return out


# speedup vs baseline: 1.0080x; 1.0080x over previous
import functools

import jax
import jax.numpy as jnp
from jax import lax
from jax.experimental import pallas as pl
from jax.experimental.pallas import tpu as pltpu
from jax.experimental.pallas import tpu_sc as plsc

_NC = 2
_NS = 16
_NW = _NC * _NS
_L = 16

_N = 10000
_E = 320000
_D = 128

_CH = 128
_NSUB = 8
_KP = 40
_NPH0 = 3
_NPH1 = 1
_K0 = _NPH0 * _KP
_K1 = _NPH1 * _KP
_TOTC = _NS * (_K0 + _K1)
_EPT = (_K0 + _K1) * _CH
_EPAD = _TOTC * _CH
_EPD = _EPAD // _NW
_ACC = 10240
_RPT = _ACC // _NS

_mesh = plsc.VectorSubcoreMesh(
    core_axis_name="c", subcore_axis_name="s", num_cores=_NC, num_subcores=_NS
)


@functools.partial(
    pl.kernel,
    out_type=jax.ShapeDtypeStruct((_NW, _ACC), jnp.float32),
    mesh=_mesh,
    scratch_types=[
        pltpu.VMEM((_EPD,), jnp.int32),
        pltpu.VMEM((_ACC,), jnp.float32),
    ],
    compiler_params=pltpu.CompilerParams(needs_layout_passes=False),
)
def _deg_kernel(dst_hbm, out_hbm, dstv, degv):
    c = lax.axis_index("c")
    s = lax.axis_index("s")
    w = c * _NS + s
    pltpu.sync_copy(dst_hbm.at[pl.ds(w * _EPD, _EPD)], dstv)

    zeros = jnp.zeros((_L,), jnp.float32)

    @pl.loop(0, _ACC // _L)
    def _zero(i):
        degv[pl.ds(i * _L, _L)] = zeros

    ones = jnp.ones((_L,), jnp.float32)

    @pl.loop(0, _EPD // _L)
    def _hist(i):
        idx = dstv[pl.ds(i * _L, _L)]
        plsc.addupdate_scatter(degv, [idx], ones)

    pltpu.sync_copy(degv, out_hbm.at[w])


def _dis_body(parts_ref, dis_ref):
    deg = jnp.sum(parts_ref[...], axis=0) + 1.0
    dis_ref[...] = lax.rsqrt(deg)


def _dis_kernel(parts):
    return pl.pallas_call(
        _dis_body,
        out_shape=jax.ShapeDtypeStruct((_ACC // 128, 128), jnp.float32),
    )(parts.reshape(_NW, _ACC // 128, 128))


def _matmul_body(x_ref, w_ref, dis_ref, g_ref):
    h = lax.dot_general(
        x_ref[...], w_ref[...],
        dimension_numbers=(((1,), (1,)), ((), ())),
        preferred_element_type=jnp.float32,
    )
    g_ref[...] = h * dis_ref[...]


def _matmul_kernel(x, w, dis):
    bm = 1000
    return pl.pallas_call(
        _matmul_body,
        grid=(_N // bm,),
        in_specs=[
            pl.BlockSpec((bm, _D), lambda i: (i, 0)),
            pl.BlockSpec((_D, _D), lambda i: (0, 0)),
            pl.BlockSpec((bm, 1), lambda i: (i, 0)),
        ],
        out_specs=pl.BlockSpec((bm, _D), lambda i: (i, 0)),
        out_shape=jax.ShapeDtypeStruct((_N, _D), jnp.float32),
    )(x, w, dis)


@functools.partial(
    pl.kernel,
    out_type=jax.ShapeDtypeStruct((_NC, _ACC, _D), jnp.float32),
    mesh=_mesh,
    scratch_types=[
        pltpu.VMEM((_KP, _CH), jnp.int32),
        pltpu.VMEM((_KP, _CH), jnp.int32),

        pltpu.VMEM((_CH, _D), jnp.float32),
        pltpu.VMEM((_CH, _D), jnp.float32),
        pltpu.VMEM_SHARED((_ACC, _D), jnp.float32),
        pltpu.SemaphoreType.DMA,
        pltpu.SemaphoreType.DMA,
    ],
)
def _scatter_kernel(g_hbm, src_hbm, dst_hbm, out_hbm,
                    srcv, dstv, buf0, buf1, acc, sem0, sem1):
    c = lax.axis_index("c")
    s = lax.axis_index("s")
    base = jnp.where(c == 0, s * _K0, _NS * _K0 + s * _K1)
    nph = jnp.where(c == 0, _NPH0, _NPH1)

    zeros = jnp.zeros((_L,), jnp.float32)

    @pl.loop(0, _CH * _D // _L)
    def _zero(i):
        r = i >> 3
        col = (i & 7) * _L
        buf0[r, pl.ds(col, _L)] = zeros

    @pl.loop(0, _RPT // _CH)
    def _zacc(i):
        pltpu.sync_copy(buf0, acc.at[pl.ds(s * _RPT + i * _CH, _CH)])

    plsc.subcore_barrier()

    def _start(j, buf, sem):
        sub = _CH // _NSUB
        for q in range(_NSUB):
            pltpu.async_copy(
                g_hbm.at[srcv.at[j, pl.ds(q * sub, sub)]],
                buf.at[pl.ds(q * sub, sub)],
                sem,
            )

    def _wait(buf, sem):
        pltpu.make_async_copy(g_hbm.at[srcv.at[0]], buf, sem).wait()

    @pl.loop(0, nph)
    def _phase(p):
        pltpu.sync_copy(src_hbm.at[pl.ds(base + p * _KP, _KP)], srcv)
        pltpu.sync_copy(dst_hbm.at[pl.ds(base + p * _KP, _KP)], dstv)

        _start(0, buf0, sem0)
        _start(1, buf1, sem1)

        @pl.loop(0, _KP // 2)
        def _edges(i):
            j0 = 2 * i
            _wait(buf0, sem0)
            pltpu.sync_copy(buf0, acc.at[dstv.at[j0]], add=True)

            @pl.when(j0 + 2 < _KP)
            def _():
                _start(j0 + 2, buf0, sem0)

            j1 = 2 * i + 1
            _wait(buf1, sem1)
            pltpu.sync_copy(buf1, acc.at[dstv.at[j1]], add=True)

            @pl.when(j1 + 2 < _KP)
            def _():
                _start(j1 + 2, buf1, sem1)

    plsc.subcore_barrier()
    pltpu.sync_copy(acc.at[pl.ds(s * _RPT, _RPT)],
                    out_hbm.at[c, pl.ds(s * _RPT, _RPT)])


def _final_body(p0_ref, p1_ref, g_ref, dis_ref, b_ref, out_ref):
    out_ref[...] = (
        dis_ref[...] * (p0_ref[...] + p1_ref[...] + g_ref[...]) + b_ref[...]
    )


def _final_kernel(p0, p1, g, dis, b):
    bm = 1000
    row = pl.BlockSpec((bm, _D), lambda i: (i, 0))
    return pl.pallas_call(
        _final_body,
        grid=(_N // bm,),
        in_specs=[
            row, row, row,
            pl.BlockSpec((bm, 1), lambda i: (i, 0)),
            pl.BlockSpec((1, _D), lambda i: (0, 0)),
        ],
        out_specs=row,
        out_shape=jax.ShapeDtypeStruct((_N, _D), jnp.float32),
    )(p0, p1, g, dis, b)


def kernel(x, edge_index, W, b):
    src = edge_index[0].astype(jnp.int32)
    dst = edge_index[1].astype(jnp.int32)
    npad = _EPAD - _E
    src_p = jnp.concatenate([src, jnp.zeros((npad,), jnp.int32)])
    dst_p = jnp.concatenate([dst, jnp.full((npad,), _N, jnp.int32)])

    deg_parts = _deg_kernel(dst_p)
    dis2d = _dis_kernel(deg_parts)
    dis = dis2d.reshape(-1)[:_N, None]
    g = _matmul_kernel(x, W, dis)
    parts = _scatter_kernel(
        g, src_p.reshape(_TOTC, _CH), dst_p.reshape(_TOTC, _CH)
    )
    out = _final_kernel(parts[0, :_N], parts[1, :_N], g, dis, b.reshape(1, _D))
    return out


# speedup vs baseline: 1.0184x; 1.0103x over previous
import functools

import jax
import jax.numpy as jnp
from jax import lax
from jax.experimental import pallas as pl
from jax.experimental.pallas import tpu as pltpu
from jax.experimental.pallas import tpu_sc as plsc

_NC = 2
_NS = 16
_NW = _NC * _NS
_L = 16

_N = 10000
_E = 320000
_D = 128

_CH = 128
_NSUB = 4
_KP = 40
_NPH0 = 3
_NPH1 = 1
_K0 = _NPH0 * _KP
_K1 = _NPH1 * _KP
_TOTC = _NS * (_K0 + _K1)
_EPT = (_K0 + _K1) * _CH
_EPAD = _TOTC * _CH
_EPD = _EPAD // _NW
_ACC = 10240
_RPT = _ACC // _NS

_mesh = plsc.VectorSubcoreMesh(
    core_axis_name="c", subcore_axis_name="s", num_cores=_NC, num_subcores=_NS
)


@functools.partial(
    pl.kernel,
    out_type=jax.ShapeDtypeStruct((_NW, _ACC), jnp.float32),
    mesh=_mesh,
    scratch_types=[
        pltpu.VMEM((_EPD,), jnp.int32),
        pltpu.VMEM((_ACC,), jnp.float32),
    ],
    compiler_params=pltpu.CompilerParams(needs_layout_passes=False),
)
def _deg_kernel(dst_hbm, out_hbm, dstv, degv):
    c = lax.axis_index("c")
    s = lax.axis_index("s")
    w = c * _NS + s
    pltpu.sync_copy(dst_hbm.at[pl.ds(w * _EPD, _EPD)], dstv)

    zeros = jnp.zeros((_L,), jnp.float32)

    @pl.loop(0, _ACC // _L)
    def _zero(i):
        degv[pl.ds(i * _L, _L)] = zeros

    ones = jnp.ones((_L,), jnp.float32)

    @pl.loop(0, _EPD // _L)
    def _hist(i):
        idx = dstv[pl.ds(i * _L, _L)]
        plsc.addupdate_scatter(degv, [idx], ones)

    pltpu.sync_copy(degv, out_hbm.at[w])


def _dis_body(parts_ref, dis_ref):
    deg = jnp.sum(parts_ref[...], axis=0) + 1.0
    dis_ref[...] = lax.rsqrt(deg)


def _dis_kernel(parts):
    return pl.pallas_call(
        _dis_body,
        out_shape=jax.ShapeDtypeStruct((_ACC // 128, 128), jnp.float32),
    )(parts.reshape(_NW, _ACC // 128, 128))


def _matmul_body(x_ref, w_ref, h_ref):
    h_ref[...] = lax.dot_general(
        x_ref[...], w_ref[...],
        dimension_numbers=(((1,), (1,)), ((), ())),
        preferred_element_type=jnp.float32,
    )


def _matmul_kernel(x, w):
    bm = 1000
    return pl.pallas_call(
        _matmul_body,
        grid=(_N // bm,),
        in_specs=[
            pl.BlockSpec((bm, _D), lambda i: (i, 0)),
            pl.BlockSpec((_D, _D), lambda i: (0, 0)),
        ],
        out_specs=pl.BlockSpec((bm, _D), lambda i: (i, 0)),
        out_shape=jax.ShapeDtypeStruct((_N, _D), jnp.float32),
    )(x, w)


def _scale_body(h_ref, dis_ref, g_ref):
    g_ref[...] = h_ref[...] * dis_ref[...]


def _scale_kernel(h, dis):
    bm = 1000
    row = pl.BlockSpec((bm, _D), lambda i: (i, 0))
    return pl.pallas_call(
        _scale_body,
        grid=(_N // bm,),
        in_specs=[row, pl.BlockSpec((bm, 1), lambda i: (i, 0))],
        out_specs=row,
        out_shape=jax.ShapeDtypeStruct((_N, _D), jnp.float32),
    )(h, dis)


@functools.partial(
    pl.kernel,
    out_type=jax.ShapeDtypeStruct((_NC, _ACC, _D), jnp.float32),
    mesh=_mesh,
    scratch_types=[
        pltpu.VMEM((_KP, _CH), jnp.int32),
        pltpu.VMEM((_KP, _CH), jnp.int32),

        pltpu.VMEM((_CH, _D), jnp.float32),
        pltpu.VMEM((_CH, _D), jnp.float32),
        pltpu.VMEM_SHARED((_ACC, _D), jnp.float32),
        pltpu.SemaphoreType.DMA,
        pltpu.SemaphoreType.DMA,
    ],
)
def _scatter_kernel(g_hbm, src_hbm, dst_hbm, out_hbm,
                    srcv, dstv, buf0, buf1, acc, sem0, sem1):
    c = lax.axis_index("c")
    s = lax.axis_index("s")
    base = jnp.where(c == 0, s * _K0, _NS * _K0 + s * _K1)
    nph = jnp.where(c == 0, _NPH0, _NPH1)

    zeros = jnp.zeros((_L,), jnp.float32)

    @pl.loop(0, _CH * _D // _L)
    def _zero(i):
        r = i >> 3
        col = (i & 7) * _L
        buf0[r, pl.ds(col, _L)] = zeros

    @pl.loop(0, _RPT // _CH)
    def _zacc(i):
        pltpu.sync_copy(buf0, acc.at[pl.ds(s * _RPT + i * _CH, _CH)])

    plsc.subcore_barrier()

    def _start(j, buf, sem):
        sub = _CH // _NSUB
        for q in range(_NSUB):
            pltpu.async_copy(
                g_hbm.at[srcv.at[j, pl.ds(q * sub, sub)]],
                buf.at[pl.ds(q * sub, sub)],
                sem,
            )

    def _wait(buf, sem):
        pltpu.make_async_copy(g_hbm.at[srcv.at[0]], buf, sem).wait()

    @pl.loop(0, nph)
    def _phase(p):
        pltpu.sync_copy(src_hbm.at[pl.ds(base + p * _KP, _KP)], srcv)
        pltpu.sync_copy(dst_hbm.at[pl.ds(base + p * _KP, _KP)], dstv)

        _start(0, buf0, sem0)
        _start(1, buf1, sem1)

        @pl.loop(0, _KP // 2)
        def _edges(i):
            j0 = 2 * i
            _wait(buf0, sem0)
            pltpu.sync_copy(buf0, acc.at[dstv.at[j0]], add=True)

            @pl.when(j0 + 2 < _KP)
            def _():
                _start(j0 + 2, buf0, sem0)

            j1 = 2 * i + 1
            _wait(buf1, sem1)
            pltpu.sync_copy(buf1, acc.at[dstv.at[j1]], add=True)

            @pl.when(j1 + 2 < _KP)
            def _():
                _start(j1 + 2, buf1, sem1)

    plsc.subcore_barrier()
    pltpu.sync_copy(acc.at[pl.ds(s * _RPT, _RPT)],
                    out_hbm.at[c, pl.ds(s * _RPT, _RPT)])


def _final_body(p0_ref, p1_ref, g_ref, dis_ref, b_ref, out_ref):
    out_ref[...] = (
        dis_ref[...] * (p0_ref[...] + p1_ref[...] + g_ref[...]) + b_ref[...]
    )


def _final_kernel(p0, p1, g, dis, b):
    bm = 1000
    row = pl.BlockSpec((bm, _D), lambda i: (i, 0))
    return pl.pallas_call(
        _final_body,
        grid=(_N // bm,),
        in_specs=[
            row, row, row,
            pl.BlockSpec((bm, 1), lambda i: (i, 0)),
            pl.BlockSpec((1, _D), lambda i: (0, 0)),
        ],
        out_specs=row,
        out_shape=jax.ShapeDtypeStruct((_N, _D), jnp.float32),
    )(p0, p1, g, dis, b)


def kernel(x, edge_index, W, b):
    src = edge_index[0].astype(jnp.int32)
    dst = edge_index[1].astype(jnp.int32)
    npad = _EPAD - _E
    src_p = jnp.concatenate([src, jnp.zeros((npad,), jnp.int32)])
    dst_p = jnp.concatenate([dst, jnp.full((npad,), _N, jnp.int32)])

    h = _matmul_kernel(x, W)
    deg_parts = _deg_kernel(dst_p)
    dis2d = _dis_kernel(deg_parts)
    dis = dis2d.reshape(-1)[:_N, None]
    g = _scale_kernel(h, dis)
    parts = _scatter_kernel(
        g, src_p.reshape(_TOTC, _CH), dst_p.reshape(_TOTC, _CH)
    )
    out = _final_kernel(parts[0, :_N], parts[1, :_N], g, dis, b.reshape(1, _D))
    return out
